# Initial kernel scaffold; baseline (speedup 1.0000x reference)
import jax
import jax.numpy as jnp
from jax.experimental import pallas as pl


def kernel(x_wav, W_shared, W_phi, b_phi, W_mu, b_mu, W_Q, W_K):
    raise NotImplementedError("write your pallas kernel here")


# speedup vs baseline: 8.3556x; 8.3556x over previous
import functools

import jax
import jax.numpy as jnp
from jax.experimental import pallas as pl

_B, _N, _T, _C = 1, 4096, 12, 128
_R = _C // 4
_D = _C // 2
_TOPK = 16
_INV_SCALE = 1.0 / float(_D) ** 0.5
_NEG = -3.0e38

_NB_QK = 512
_NB_G = 256


def _qk_kernel(x_ref, ws_ref, wphi_ref, bphi_ref, wmu_ref, wq_ref, wk_ref,
               q_ref, k_ref):
    ws = ws_ref[:]
    xs = [x_ref[:, t * _C:(t + 1) * _C] for t in range(_T)]
    avg = xs[0]
    for t in range(1, _T):
        avg = avg + xs[t]
    avg = avg / float(_T)
    z_ca = jax.lax.dot_general(avg, ws, (((1,), (1,)), ((), ())),
                               preferred_element_type=jnp.float32)
    z_ca = jnp.maximum(z_ca, 0.0)
    w = jax.lax.dot_general(z_ca, wphi_ref[:], (((1,), (1,)), ((), ())),
                            preferred_element_type=jnp.float32)
    w = jax.nn.sigmoid(w + bphi_ref[:])
    wmu = wmu_ref[:].astype(jnp.bfloat16).astype(jnp.float32)
    xh = []
    mus = []
    for t in range(_T):
        xh_t = xs[t] * w
        xh.append(xh_t)
        z_t = jax.lax.dot_general(xh_t, ws, (((1,), (1,)), ((), ())),
                                  preferred_element_type=jnp.float32)
        z_t = jnp.maximum(z_t, 0.0)
        z_tb = z_t.astype(jnp.bfloat16).astype(jnp.float32)
        mus.append(jnp.sum(z_tb * wmu, axis=1, keepdims=True))
    m = mus[0]
    for t in range(1, _T):
        m = jnp.maximum(m, mus[t])
    es = [jnp.exp(mu_t - m) for mu_t in mus]
    s = es[0]
    for t in range(1, _T):
        s = s + es[t]
    xc = None
    for t in range(_T):
        contrib = xh[t] * (es[t] / s)
        xc = contrib if xc is None else xc + contrib
    q = jax.lax.dot_general(xc, wq_ref[:], (((1,), (1,)), ((), ())),
                            preferred_element_type=jnp.float32)
    q_ref[:] = q * _INV_SCALE
    k_ref[:] = jax.lax.dot_general(xc, wk_ref[:], (((1,), (1,)), ((), ())),
                                   preferred_element_type=jnp.float32)


def _topk_masked(s):
    iota = jax.lax.broadcasted_iota(jnp.int32, s.shape, 1)
    work = s
    out = jnp.zeros_like(s)
    for _ in range(_TOPK):
        m = jnp.max(work, axis=1, keepdims=True)
        idx = jnp.min(jnp.where(work == m, iota, _N), axis=1, keepdims=True)
        hit = iota == idx
        out = jnp.where(hit, work, out)
        work = jnp.where(hit, -1.0, work)
    return out


def _graph_kernel(qb_ref, kb_ref, qa_ref, ka_ref, sf_ref, sb_ref):
    qb = qb_ref[:].astype(jnp.bfloat16)
    kb = kb_ref[:].astype(jnp.bfloat16)
    qa = qa_ref[:].astype(jnp.bfloat16)
    ka = ka_ref[:].astype(jnp.bfloat16)
    ef = jax.lax.dot_general(qb, ka, (((1,), (1,)), ((), ())),
                             preferred_element_type=jnp.float32)
    sf_ref[:] = _topk_masked(jax.nn.sigmoid(ef))
    eb = jax.lax.dot_general(kb, qa, (((1,), (1,)), ((), ())),
                             preferred_element_type=jnp.float32)
    sb_ref[:] = _topk_masked(jax.nn.sigmoid(eb))


@functools.partial(jax.jit, static_argnames=("interpret",))
def _run(x_wav, W_shared, W_phi, b_phi, W_mu, W_Q, W_K, interpret=False):
    x2 = x_wav.reshape(_N, _T * _C)
    bphi2 = b_phi.reshape(1, _C)
    wmu2 = W_mu.reshape(1, _R)

    q, k = pl.pallas_call(
        _qk_kernel,
        grid=(_N // _NB_QK,),
        in_specs=[
            pl.BlockSpec((_NB_QK, _T * _C), lambda i: (i, 0)),
            pl.BlockSpec((_R, _C), lambda i: (0, 0)),
            pl.BlockSpec((_C, _R), lambda i: (0, 0)),
            pl.BlockSpec((1, _C), lambda i: (0, 0)),
            pl.BlockSpec((1, _R), lambda i: (0, 0)),
            pl.BlockSpec((_D, _C), lambda i: (0, 0)),
            pl.BlockSpec((_D, _C), lambda i: (0, 0)),
        ],
        out_specs=[
            pl.BlockSpec((_NB_QK, _D), lambda i: (i, 0)),
            pl.BlockSpec((_NB_QK, _D), lambda i: (i, 0)),
        ],
        out_shape=[
            jax.ShapeDtypeStruct((_N, _D), jnp.float32),
            jax.ShapeDtypeStruct((_N, _D), jnp.float32),
        ],
        interpret=interpret,
    )(x2, W_shared, W_phi, bphi2, wmu2, W_Q, W_K)

    sf, sb = pl.pallas_call(
        _graph_kernel,
        grid=(_N // _NB_G,),
        in_specs=[
            pl.BlockSpec((_NB_G, _D), lambda i: (i, 0)),
            pl.BlockSpec((_NB_G, _D), lambda i: (i, 0)),
            pl.BlockSpec((_N, _D), lambda i: (0, 0)),
            pl.BlockSpec((_N, _D), lambda i: (0, 0)),
        ],
        out_specs=[
            pl.BlockSpec((_NB_G, _N), lambda i: (i, 0)),
            pl.BlockSpec((_NB_G, _N), lambda i: (i, 0)),
        ],
        out_shape=[
            jax.ShapeDtypeStruct((_N, _N), jnp.float32),
            jax.ShapeDtypeStruct((_N, _N), jnp.float32),
        ],
        interpret=interpret,
    )(q, k, q, k)

    return sf[None], sb[None]


def kernel(x_wav, W_shared, W_phi, b_phi, W_mu, b_mu, W_Q, W_K):
    del b_mu
    return _run(x_wav, W_shared, W_phi, b_phi, W_mu, W_Q, W_K)


# speedup vs baseline: 10.2038x; 1.2212x over previous
import functools

import jax
import jax.numpy as jnp
from jax.experimental import pallas as pl

_B, _N, _T, _C = 1, 4096, 12, 128
_R = _C // 4
_D = _C // 2
_TOPK = 16
_INV_SCALE = 1.0 / float(_D) ** 0.5
_NEG = -3.0e38

_NB_QK = 512
_NB_G = 256


def _qk_kernel(x_ref, ws_ref, wphi_ref, bphi_ref, wmu_ref, wq_ref, wk_ref,
               q_ref, k_ref):
    ws = ws_ref[:]
    xs = [x_ref[:, t * _C:(t + 1) * _C] for t in range(_T)]
    avg = xs[0]
    for t in range(1, _T):
        avg = avg + xs[t]
    avg = avg / float(_T)
    z_ca = jax.lax.dot_general(avg, ws, (((1,), (1,)), ((), ())),
                               preferred_element_type=jnp.float32)
    z_ca = jnp.maximum(z_ca, 0.0)
    w = jax.lax.dot_general(z_ca, wphi_ref[:], (((1,), (1,)), ((), ())),
                            preferred_element_type=jnp.float32)
    w = jax.nn.sigmoid(w + bphi_ref[:])
    wmu = wmu_ref[:].astype(jnp.bfloat16).astype(jnp.float32)
    xh = []
    mus = []
    for t in range(_T):
        xh_t = xs[t] * w
        xh.append(xh_t)
        z_t = jax.lax.dot_general(xh_t, ws, (((1,), (1,)), ((), ())),
                                  preferred_element_type=jnp.float32)
        z_t = jnp.maximum(z_t, 0.0)
        z_tb = z_t.astype(jnp.bfloat16).astype(jnp.float32)
        mus.append(jnp.sum(z_tb * wmu, axis=1, keepdims=True))
    m = mus[0]
    for t in range(1, _T):
        m = jnp.maximum(m, mus[t])
    es = [jnp.exp(mu_t - m) for mu_t in mus]
    s = es[0]
    for t in range(1, _T):
        s = s + es[t]
    xc = None
    for t in range(_T):
        contrib = xh[t] * (es[t] / s)
        xc = contrib if xc is None else xc + contrib
    q = jax.lax.dot_general(xc, wq_ref[:], (((1,), (1,)), ((), ())),
                            preferred_element_type=jnp.float32)
    q_ref[:] = q * _INV_SCALE
    k_ref[:] = jax.lax.dot_general(xc, wk_ref[:], (((1,), (1,)), ((), ())),
                                   preferred_element_type=jnp.float32)


def _topk_masked(s):
    iota = jax.lax.broadcasted_iota(jnp.int32, s.shape, 1)
    work = s
    for _ in range(_TOPK):
        m = jnp.max(work, axis=1, keepdims=True)
        cand = jnp.where(work == m, iota, _N)
        idx = jnp.min(cand, axis=1, keepdims=True)
        work = jnp.where(cand == idx, -1.0, work)
    return jnp.where(work < 0.0, s, 0.0)


def _graph_kernel(qb_ref, kb_ref, qa_ref, ka_ref, sf_ref, sb_ref):
    qb = qb_ref[:].astype(jnp.bfloat16)
    kb = kb_ref[:].astype(jnp.bfloat16)
    qa = qa_ref[:].astype(jnp.bfloat16)
    ka = ka_ref[:].astype(jnp.bfloat16)
    ef = jax.lax.dot_general(qb, ka, (((1,), (1,)), ((), ())),
                             preferred_element_type=jnp.float32)
    sf_ref[:] = _topk_masked(jax.nn.sigmoid(ef))
    eb = jax.lax.dot_general(kb, qa, (((1,), (1,)), ((), ())),
                             preferred_element_type=jnp.float32)
    sb_ref[:] = _topk_masked(jax.nn.sigmoid(eb))


@functools.partial(jax.jit, static_argnames=("interpret",))
def _run(x_wav, W_shared, W_phi, b_phi, W_mu, W_Q, W_K, interpret=False):
    x2 = x_wav.reshape(_N, _T * _C)
    bphi2 = b_phi.reshape(1, _C)
    wmu2 = W_mu.reshape(1, _R)

    q, k = pl.pallas_call(
        _qk_kernel,
        grid=(_N // _NB_QK,),
        in_specs=[
            pl.BlockSpec((_NB_QK, _T * _C), lambda i: (i, 0)),
            pl.BlockSpec((_R, _C), lambda i: (0, 0)),
            pl.BlockSpec((_C, _R), lambda i: (0, 0)),
            pl.BlockSpec((1, _C), lambda i: (0, 0)),
            pl.BlockSpec((1, _R), lambda i: (0, 0)),
            pl.BlockSpec((_D, _C), lambda i: (0, 0)),
            pl.BlockSpec((_D, _C), lambda i: (0, 0)),
        ],
        out_specs=[
            pl.BlockSpec((_NB_QK, _D), lambda i: (i, 0)),
            pl.BlockSpec((_NB_QK, _D), lambda i: (i, 0)),
        ],
        out_shape=[
            jax.ShapeDtypeStruct((_N, _D), jnp.float32),
            jax.ShapeDtypeStruct((_N, _D), jnp.float32),
        ],
        interpret=interpret,
    )(x2, W_shared, W_phi, bphi2, wmu2, W_Q, W_K)

    sf, sb = pl.pallas_call(
        _graph_kernel,
        grid=(_N // _NB_G,),
        in_specs=[
            pl.BlockSpec((_NB_G, _D), lambda i: (i, 0)),
            pl.BlockSpec((_NB_G, _D), lambda i: (i, 0)),
            pl.BlockSpec((_N, _D), lambda i: (0, 0)),
            pl.BlockSpec((_N, _D), lambda i: (0, 0)),
        ],
        out_specs=[
            pl.BlockSpec((_NB_G, _N), lambda i: (i, 0)),
            pl.BlockSpec((_NB_G, _N), lambda i: (i, 0)),
        ],
        out_shape=[
            jax.ShapeDtypeStruct((_N, _N), jnp.float32),
            jax.ShapeDtypeStruct((_N, _N), jnp.float32),
        ],
        interpret=interpret,
    )(q, k, q, k)

    return sf[None], sb[None]


def kernel(x_wav, W_shared, W_phi, b_phi, W_mu, b_mu, W_Q, W_K):
    del b_mu
    return _run(x_wav, W_shared, W_phi, b_phi, W_mu, W_Q, W_K)


# speedup vs baseline: 20.6377x; 2.0226x over previous
import functools

import jax
import jax.numpy as jnp
from jax.experimental import pallas as pl

_B, _N, _T, _C = 1, 4096, 12, 128
_R = _C // 4
_D = _C // 2
_TOPK = 16
_INV_SCALE = 1.0 / float(_D) ** 0.5
_NEG = -3.0e38

_NB_QK = 512
_NB_G = 256


def _qk_kernel(x_ref, ws_ref, wphi_ref, bphi_ref, wmu_ref, wq_ref, wk_ref,
               q_ref, k_ref):
    ws = ws_ref[:]
    xs = [x_ref[:, t * _C:(t + 1) * _C] for t in range(_T)]
    avg = xs[0]
    for t in range(1, _T):
        avg = avg + xs[t]
    avg = avg / float(_T)
    z_ca = jax.lax.dot_general(avg, ws, (((1,), (1,)), ((), ())),
                               preferred_element_type=jnp.float32)
    z_ca = jnp.maximum(z_ca, 0.0)
    w = jax.lax.dot_general(z_ca, wphi_ref[:], (((1,), (1,)), ((), ())),
                            preferred_element_type=jnp.float32)
    w = jax.nn.sigmoid(w + bphi_ref[:])
    wmu = wmu_ref[:].astype(jnp.bfloat16).astype(jnp.float32)
    xh = []
    mus = []
    for t in range(_T):
        xh_t = xs[t] * w
        xh.append(xh_t)
        z_t = jax.lax.dot_general(xh_t, ws, (((1,), (1,)), ((), ())),
                                  preferred_element_type=jnp.float32)
        z_t = jnp.maximum(z_t, 0.0)
        z_tb = z_t.astype(jnp.bfloat16).astype(jnp.float32)
        mus.append(jnp.sum(z_tb * wmu, axis=1, keepdims=True))
    m = mus[0]
    for t in range(1, _T):
        m = jnp.maximum(m, mus[t])
    es = [jnp.exp(mu_t - m) for mu_t in mus]
    s = es[0]
    for t in range(1, _T):
        s = s + es[t]
    xc = None
    for t in range(_T):
        contrib = xh[t] * (es[t] / s)
        xc = contrib if xc is None else xc + contrib
    q = jax.lax.dot_general(xc, wq_ref[:], (((1,), (1,)), ((), ())),
                            preferred_element_type=jnp.float32)
    q_ref[:] = q * _INV_SCALE
    k_ref[:] = jax.lax.dot_general(xc, wk_ref[:], (((1,), (1,)), ((), ())),
                                   preferred_element_type=jnp.float32)


_CW = 128
_NCH = _N // _CW


def _topk_masked(e, out_ref):
    work = e
    m = None
    for i in range(_TOPK):
        m = jnp.max(work, axis=1, keepdims=True)
        if i != _TOPK - 1:
            work = jnp.where(work == m, _NEG, work)
    s = jax.nn.sigmoid(e)
    sv16 = jax.nn.sigmoid(m)
    gt = (s > sv16).astype(jnp.float32)
    eq = (s == sv16).astype(jnp.float32)
    need = 16.0 - jnp.sum(gt, axis=1, keepdims=True)
    r_i = jax.lax.broadcasted_iota(jnp.int32, (_CW, _CW), 0)
    c_i = jax.lax.broadcasted_iota(jnp.int32, (_CW, _CW), 1)
    triu = (r_i < c_i).astype(jnp.bfloat16)
    off = jnp.zeros_like(need)
    for c in range(_NCH):
        sl = slice(c * _CW, (c + 1) * _CW)
        eq_c = eq[:, sl]
        rank_c = jax.lax.dot_general(eq_c.astype(jnp.bfloat16), triu,
                                     (((1,), (0,)), ((), ())),
                                     preferred_element_type=jnp.float32)
        keep_c = (gt[:, sl] + eq_c * (rank_c + off < need)) > 0.0
        out_ref[:, sl] = jnp.where(keep_c, s[:, sl], 0.0)
        off = off + jnp.sum(eq_c, axis=1, keepdims=True)


def _graph_kernel(qb_ref, kb_ref, qa_ref, ka_ref, sf_ref, sb_ref):
    qb = qb_ref[:].astype(jnp.bfloat16)
    kb = kb_ref[:].astype(jnp.bfloat16)
    qa = qa_ref[:].astype(jnp.bfloat16)
    ka = ka_ref[:].astype(jnp.bfloat16)
    ef = jax.lax.dot_general(qb, ka, (((1,), (1,)), ((), ())),
                             preferred_element_type=jnp.float32)
    _topk_masked(ef, sf_ref)
    eb = jax.lax.dot_general(kb, qa, (((1,), (1,)), ((), ())),
                             preferred_element_type=jnp.float32)
    _topk_masked(eb, sb_ref)


@functools.partial(jax.jit, static_argnames=("interpret",))
def _run(x_wav, W_shared, W_phi, b_phi, W_mu, W_Q, W_K, interpret=False):
    x2 = x_wav.reshape(_N, _T * _C)
    bphi2 = b_phi.reshape(1, _C)
    wmu2 = W_mu.reshape(1, _R)

    q, k = pl.pallas_call(
        _qk_kernel,
        grid=(_N // _NB_QK,),
        in_specs=[
            pl.BlockSpec((_NB_QK, _T * _C), lambda i: (i, 0)),
            pl.BlockSpec((_R, _C), lambda i: (0, 0)),
            pl.BlockSpec((_C, _R), lambda i: (0, 0)),
            pl.BlockSpec((1, _C), lambda i: (0, 0)),
            pl.BlockSpec((1, _R), lambda i: (0, 0)),
            pl.BlockSpec((_D, _C), lambda i: (0, 0)),
            pl.BlockSpec((_D, _C), lambda i: (0, 0)),
        ],
        out_specs=[
            pl.BlockSpec((_NB_QK, _D), lambda i: (i, 0)),
            pl.BlockSpec((_NB_QK, _D), lambda i: (i, 0)),
        ],
        out_shape=[
            jax.ShapeDtypeStruct((_N, _D), jnp.float32),
            jax.ShapeDtypeStruct((_N, _D), jnp.float32),
        ],
        interpret=interpret,
    )(x2, W_shared, W_phi, bphi2, wmu2, W_Q, W_K)

    sf, sb = pl.pallas_call(
        _graph_kernel,
        grid=(_N // _NB_G,),
        in_specs=[
            pl.BlockSpec((_NB_G, _D), lambda i: (i, 0)),
            pl.BlockSpec((_NB_G, _D), lambda i: (i, 0)),
            pl.BlockSpec((_N, _D), lambda i: (0, 0)),
            pl.BlockSpec((_N, _D), lambda i: (0, 0)),
        ],
        out_specs=[
            pl.BlockSpec((_NB_G, _N), lambda i: (i, 0)),
            pl.BlockSpec((_NB_G, _N), lambda i: (i, 0)),
        ],
        out_shape=[
            jax.ShapeDtypeStruct((_N, _N), jnp.float32),
            jax.ShapeDtypeStruct((_N, _N), jnp.float32),
        ],
        interpret=interpret,
    )(q, k, q, k)

    return sf[None], sb[None]


def kernel(x_wav, W_shared, W_phi, b_phi, W_mu, b_mu, W_Q, W_K):
    del b_mu
    return _run(x_wav, W_shared, W_phi, b_phi, W_mu, W_Q, W_K)


# speedup vs baseline: 20.7132x; 1.0037x over previous
import functools

import jax
import jax.numpy as jnp
from jax.experimental import pallas as pl

_B, _N, _T, _C = 1, 4096, 12, 128
_R = _C // 4
_D = _C // 2
_TOPK = 16
_INV_SCALE = 1.0 / float(_D) ** 0.5
_NEG = -3.0e38

_NB_QK = 512
_NB_G = 256


def _qk_kernel(x_ref, ws_ref, wphi_ref, bphi_ref, wmu_ref, wq_ref, wk_ref,
               q_ref, k_ref):
    ws = ws_ref[:]
    xs = [x_ref[:, t * _C:(t + 1) * _C] for t in range(_T)]
    avg = xs[0]
    for t in range(1, _T):
        avg = avg + xs[t]
    avg = avg / float(_T)
    z_ca = jax.lax.dot_general(avg, ws, (((1,), (1,)), ((), ())),
                               preferred_element_type=jnp.float32)
    z_ca = jnp.maximum(z_ca, 0.0)
    w = jax.lax.dot_general(z_ca, wphi_ref[:], (((1,), (1,)), ((), ())),
                            preferred_element_type=jnp.float32)
    w = jax.nn.sigmoid(w + bphi_ref[:])
    wmu = wmu_ref[:].astype(jnp.bfloat16).astype(jnp.float32)
    xh = []
    mus = []
    for t in range(_T):
        xh_t = xs[t] * w
        xh.append(xh_t)
        z_t = jax.lax.dot_general(xh_t, ws, (((1,), (1,)), ((), ())),
                                  preferred_element_type=jnp.float32)
        z_t = jnp.maximum(z_t, 0.0)
        z_tb = z_t.astype(jnp.bfloat16).astype(jnp.float32)
        mus.append(jnp.sum(z_tb * wmu, axis=1, keepdims=True))
    m = mus[0]
    for t in range(1, _T):
        m = jnp.maximum(m, mus[t])
    es = [jnp.exp(mu_t - m) for mu_t in mus]
    s = es[0]
    for t in range(1, _T):
        s = s + es[t]
    xc = None
    for t in range(_T):
        contrib = xh[t] * (es[t] / s)
        xc = contrib if xc is None else xc + contrib
    q = jax.lax.dot_general(xc, wq_ref[:], (((1,), (1,)), ((), ())),
                            preferred_element_type=jnp.float32)
    q_ref[:] = q * _INV_SCALE
    k_ref[:] = jax.lax.dot_general(xc, wk_ref[:], (((1,), (1,)), ((), ())),
                                   preferred_element_type=jnp.float32)


_CW = 128
_NCH = _N // _CW


def _topk_masked(e, m, out_ref):
    s = jax.nn.sigmoid(e)
    sv16 = jax.nn.sigmoid(m)
    gt = (s > sv16).astype(jnp.float32)
    eq = (s == sv16).astype(jnp.float32)
    need = 16.0 - jnp.sum(gt, axis=1, keepdims=True)
    r_i = jax.lax.broadcasted_iota(jnp.int32, (_CW, _CW), 0)
    c_i = jax.lax.broadcasted_iota(jnp.int32, (_CW, _CW), 1)
    triu = (r_i < c_i).astype(jnp.bfloat16)
    off = jnp.zeros_like(need)
    for c in range(_NCH):
        sl = slice(c * _CW, (c + 1) * _CW)
        eq_c = eq[:, sl]
        rank_c = jax.lax.dot_general(eq_c.astype(jnp.bfloat16), triu,
                                     (((1,), (0,)), ((), ())),
                                     preferred_element_type=jnp.float32)
        keep_c = (gt[:, sl] + eq_c * (rank_c + off < need)) > 0.0
        out_ref[:, sl] = jnp.where(keep_c, s[:, sl], 0.0)
        off = off + jnp.sum(eq_c, axis=1, keepdims=True)


def _graph_kernel(qb_ref, kb_ref, qa_ref, ka_ref, sf_ref, sb_ref):
    qb = qb_ref[:].astype(jnp.bfloat16)
    kb = kb_ref[:].astype(jnp.bfloat16)
    qa = qa_ref[:].astype(jnp.bfloat16)
    ka = ka_ref[:].astype(jnp.bfloat16)
    ef = jax.lax.dot_general(qb, ka, (((1,), (1,)), ((), ())),
                             preferred_element_type=jnp.float32)
    eb = jax.lax.dot_general(kb, qa, (((1,), (1,)), ((), ())),
                             preferred_element_type=jnp.float32)
    wf, wb = ef, eb
    mf = mb = None
    for i in range(_TOPK):
        mf = jnp.max(wf, axis=1, keepdims=True)
        mb = jnp.max(wb, axis=1, keepdims=True)
        if i != _TOPK - 1:
            wf = jnp.where(wf == mf, _NEG, wf)
            wb = jnp.where(wb == mb, _NEG, wb)
    _topk_masked(ef, mf, sf_ref)
    _topk_masked(eb, mb, sb_ref)


@functools.partial(jax.jit, static_argnames=("interpret",))
def _run(x_wav, W_shared, W_phi, b_phi, W_mu, W_Q, W_K, interpret=False):
    x2 = x_wav.reshape(_N, _T * _C)
    bphi2 = b_phi.reshape(1, _C)
    wmu2 = W_mu.reshape(1, _R)

    q, k = pl.pallas_call(
        _qk_kernel,
        grid=(_N // _NB_QK,),
        in_specs=[
            pl.BlockSpec((_NB_QK, _T * _C), lambda i: (i, 0)),
            pl.BlockSpec((_R, _C), lambda i: (0, 0)),
            pl.BlockSpec((_C, _R), lambda i: (0, 0)),
            pl.BlockSpec((1, _C), lambda i: (0, 0)),
            pl.BlockSpec((1, _R), lambda i: (0, 0)),
            pl.BlockSpec((_D, _C), lambda i: (0, 0)),
            pl.BlockSpec((_D, _C), lambda i: (0, 0)),
        ],
        out_specs=[
            pl.BlockSpec((_NB_QK, _D), lambda i: (i, 0)),
            pl.BlockSpec((_NB_QK, _D), lambda i: (i, 0)),
        ],
        out_shape=[
            jax.ShapeDtypeStruct((_N, _D), jnp.float32),
            jax.ShapeDtypeStruct((_N, _D), jnp.float32),
        ],
        interpret=interpret,
    )(x2, W_shared, W_phi, bphi2, wmu2, W_Q, W_K)

    sf, sb = pl.pallas_call(
        _graph_kernel,
        grid=(_N // _NB_G,),
        in_specs=[
            pl.BlockSpec((_NB_G, _D), lambda i: (i, 0)),
            pl.BlockSpec((_NB_G, _D), lambda i: (i, 0)),
            pl.BlockSpec((_N, _D), lambda i: (0, 0)),
            pl.BlockSpec((_N, _D), lambda i: (0, 0)),
        ],
        out_specs=[
            pl.BlockSpec((_NB_G, _N), lambda i: (i, 0)),
            pl.BlockSpec((_NB_G, _N), lambda i: (i, 0)),
        ],
        out_shape=[
            jax.ShapeDtypeStruct((_N, _N), jnp.float32),
            jax.ShapeDtypeStruct((_N, _N), jnp.float32),
        ],
        interpret=interpret,
    )(q, k, q, k)

    return sf[None], sb[None]


def kernel(x_wav, W_shared, W_phi, b_phi, W_mu, b_mu, W_Q, W_K):
    del b_mu
    return _run(x_wav, W_shared, W_phi, b_phi, W_mu, W_Q, W_K)


# speedup vs baseline: 29.0586x; 1.4029x over previous
import functools

import jax
import jax.numpy as jnp
from jax.experimental import pallas as pl

_B, _N, _T, _C = 1, 4096, 12, 128
_R = _C // 4
_D = _C // 2
_TOPK = 16
_INV_SCALE = 1.0 / float(_D) ** 0.5
_NEG = -3.0e38

_NB_QK = 512
_NB_G = 256


def _qk_kernel(x_ref, ws_ref, wphi_ref, bphi_ref, wmu_ref, wq_ref, wk_ref,
               q_ref, k_ref):
    ws = ws_ref[:]
    xs = [x_ref[:, t * _C:(t + 1) * _C] for t in range(_T)]
    avg = xs[0]
    for t in range(1, _T):
        avg = avg + xs[t]
    avg = avg / float(_T)
    z_ca = jax.lax.dot_general(avg, ws, (((1,), (1,)), ((), ())),
                               preferred_element_type=jnp.float32)
    z_ca = jnp.maximum(z_ca, 0.0)
    w = jax.lax.dot_general(z_ca, wphi_ref[:], (((1,), (1,)), ((), ())),
                            preferred_element_type=jnp.float32)
    w = jax.nn.sigmoid(w + bphi_ref[:])
    wmu = wmu_ref[:].astype(jnp.bfloat16).astype(jnp.float32)
    xh = []
    mus = []
    for t in range(_T):
        xh_t = xs[t] * w
        xh.append(xh_t)
        z_t = jax.lax.dot_general(xh_t, ws, (((1,), (1,)), ((), ())),
                                  preferred_element_type=jnp.float32)
        z_t = jnp.maximum(z_t, 0.0)
        z_tb = z_t.astype(jnp.bfloat16).astype(jnp.float32)
        mus.append(jnp.sum(z_tb * wmu, axis=1, keepdims=True))
    m = mus[0]
    for t in range(1, _T):
        m = jnp.maximum(m, mus[t])
    es = [jnp.exp(mu_t - m) for mu_t in mus]
    s = es[0]
    for t in range(1, _T):
        s = s + es[t]
    xc = None
    for t in range(_T):
        contrib = xh[t] * (es[t] / s)
        xc = contrib if xc is None else xc + contrib
    q = jax.lax.dot_general(xc, wq_ref[:], (((1,), (1,)), ((), ())),
                            preferred_element_type=jnp.float32)
    q_ref[:] = q * _INV_SCALE
    k_ref[:] = jax.lax.dot_general(xc, wk_ref[:], (((1,), (1,)), ((), ())),
                                   preferred_element_type=jnp.float32)


_CW = 128
_NCH = _N // _CW


def _topk_masked(e, m, out_ref):
    s = jax.nn.sigmoid(e)
    sv16 = jax.nn.sigmoid(m)
    gt = (s > sv16).astype(jnp.float32)
    eq = (s == sv16).astype(jnp.float32)
    need = 16.0 - jnp.sum(gt, axis=1, keepdims=True)
    r_i = jax.lax.broadcasted_iota(jnp.int32, (_CW, _CW), 0)
    c_i = jax.lax.broadcasted_iota(jnp.int32, (_CW, _CW), 1)
    triu = (r_i < c_i).astype(jnp.bfloat16)
    off = jnp.zeros_like(need)
    for c in range(_NCH):
        sl = slice(c * _CW, (c + 1) * _CW)
        eq_c = eq[:, sl]
        rank_c = jax.lax.dot_general(eq_c.astype(jnp.bfloat16), triu,
                                     (((1,), (0,)), ((), ())),
                                     preferred_element_type=jnp.float32)
        keep_c = (gt[:, sl] + eq_c * (rank_c + off < need)) > 0.0
        out_ref[:, sl] = jnp.where(keep_c, s[:, sl], 0.0)
        off = off + jnp.sum(eq_c, axis=1, keepdims=True)


def _graph_kernel(qb_ref, kb_ref, qa_ref, ka_ref, sf_ref, sb_ref):
    qb = qb_ref[:].astype(jnp.bfloat16)
    kb = kb_ref[:].astype(jnp.bfloat16)
    qa = qa_ref[:].astype(jnp.bfloat16)
    ka = ka_ref[:].astype(jnp.bfloat16)
    ef = jax.lax.dot_general(qb, ka, (((1,), (1,)), ((), ())),
                             preferred_element_type=jnp.float32)
    eb = jax.lax.dot_general(kb, qa, (((1,), (1,)), ((), ())),
                             preferred_element_type=jnp.float32)
    _topk_masked(ef, _v16(ef), sf_ref)
    _topk_masked(eb, _v16(eb), sb_ref)


_LVL = 5


def _v16(e):
    work = [e[:, c * _CW:(c + 1) * _CW] for c in range(_NCH)]
    levels = []
    for lvl in range(_LVL):
        if lvl > 0:
            prev = levels[-1]
            work = [jnp.where(wc == prev, _NEG, wc) for wc in work]
        m = work[0]
        for c in range(1, _NCH):
            m = jnp.maximum(m, work[c])
        levels.append(m)
    u = jnp.concatenate(levels, axis=1)
    v = None
    for i in range(_TOPK):
        v = jnp.max(u, axis=1, keepdims=True)
        if i != _TOPK - 1:
            u = jnp.where(u == v, _NEG, u)
    return v


@functools.partial(jax.jit, static_argnames=("interpret",))
def _run(x_wav, W_shared, W_phi, b_phi, W_mu, W_Q, W_K, interpret=False):
    x2 = x_wav.reshape(_N, _T * _C)
    bphi2 = b_phi.reshape(1, _C)
    wmu2 = W_mu.reshape(1, _R)

    q, k = pl.pallas_call(
        _qk_kernel,
        grid=(_N // _NB_QK,),
        in_specs=[
            pl.BlockSpec((_NB_QK, _T * _C), lambda i: (i, 0)),
            pl.BlockSpec((_R, _C), lambda i: (0, 0)),
            pl.BlockSpec((_C, _R), lambda i: (0, 0)),
            pl.BlockSpec((1, _C), lambda i: (0, 0)),
            pl.BlockSpec((1, _R), lambda i: (0, 0)),
            pl.BlockSpec((_D, _C), lambda i: (0, 0)),
            pl.BlockSpec((_D, _C), lambda i: (0, 0)),
        ],
        out_specs=[
            pl.BlockSpec((_NB_QK, _D), lambda i: (i, 0)),
            pl.BlockSpec((_NB_QK, _D), lambda i: (i, 0)),
        ],
        out_shape=[
            jax.ShapeDtypeStruct((_N, _D), jnp.float32),
            jax.ShapeDtypeStruct((_N, _D), jnp.float32),
        ],
        interpret=interpret,
    )(x2, W_shared, W_phi, bphi2, wmu2, W_Q, W_K)

    sf, sb = pl.pallas_call(
        _graph_kernel,
        grid=(_N // _NB_G,),
        in_specs=[
            pl.BlockSpec((_NB_G, _D), lambda i: (i, 0)),
            pl.BlockSpec((_NB_G, _D), lambda i: (i, 0)),
            pl.BlockSpec((_N, _D), lambda i: (0, 0)),
            pl.BlockSpec((_N, _D), lambda i: (0, 0)),
        ],
        out_specs=[
            pl.BlockSpec((_NB_G, _N), lambda i: (i, 0)),
            pl.BlockSpec((_NB_G, _N), lambda i: (i, 0)),
        ],
        out_shape=[
            jax.ShapeDtypeStruct((_N, _N), jnp.float32),
            jax.ShapeDtypeStruct((_N, _N), jnp.float32),
        ],
        interpret=interpret,
    )(q, k, q, k)

    return sf[None], sb[None]


def kernel(x_wav, W_shared, W_phi, b_phi, W_mu, b_mu, W_Q, W_K):
    del b_mu
    return _run(x_wav, W_shared, W_phi, b_phi, W_mu, W_Q, W_K)


# speedup vs baseline: 29.1190x; 1.0021x over previous
import functools

import jax
import jax.numpy as jnp
from jax.experimental import pallas as pl

_B, _N, _T, _C = 1, 4096, 12, 128
_R = _C // 4
_D = _C // 2
_TOPK = 16
_INV_SCALE = 1.0 / float(_D) ** 0.5
_NEG = -3.0e38

_NB_QK = 512
_NB_G = 256


def _qk_kernel(x_ref, ws_ref, wphi_ref, bphi_ref, wmu_ref, wq_ref, wk_ref,
               q_ref, k_ref):
    ws = ws_ref[:]
    xs = [x_ref[:, t * _C:(t + 1) * _C] for t in range(_T)]
    avg = xs[0]
    for t in range(1, _T):
        avg = avg + xs[t]
    avg = avg / float(_T)
    z_ca = jax.lax.dot_general(avg, ws, (((1,), (1,)), ((), ())),
                               preferred_element_type=jnp.float32)
    z_ca = jnp.maximum(z_ca, 0.0)
    w = jax.lax.dot_general(z_ca, wphi_ref[:], (((1,), (1,)), ((), ())),
                            preferred_element_type=jnp.float32)
    w = jax.nn.sigmoid(w + bphi_ref[:])
    wmu = wmu_ref[:].astype(jnp.bfloat16).astype(jnp.float32)
    xh = []
    mus = []
    for t in range(_T):
        xh_t = xs[t] * w
        xh.append(xh_t)
        z_t = jax.lax.dot_general(xh_t, ws, (((1,), (1,)), ((), ())),
                                  preferred_element_type=jnp.float32)
        z_t = jnp.maximum(z_t, 0.0)
        z_tb = z_t.astype(jnp.bfloat16).astype(jnp.float32)
        mus.append(jnp.sum(z_tb * wmu, axis=1, keepdims=True))
    m = mus[0]
    for t in range(1, _T):
        m = jnp.maximum(m, mus[t])
    es = [jnp.exp(mu_t - m) for mu_t in mus]
    s = es[0]
    for t in range(1, _T):
        s = s + es[t]
    xc = None
    for t in range(_T):
        contrib = xh[t] * (es[t] / s)
        xc = contrib if xc is None else xc + contrib
    q = jax.lax.dot_general(xc, wq_ref[:], (((1,), (1,)), ((), ())),
                            preferred_element_type=jnp.float32)
    q_ref[:] = q * _INV_SCALE
    k_ref[:] = jax.lax.dot_general(xc, wk_ref[:], (((1,), (1,)), ((), ())),
                                   preferred_element_type=jnp.float32)


_CW = 128
_NCH = _N // _CW


def _topk_masked(e, m, out_ref):
    s = jax.nn.sigmoid(e)
    sv16 = jax.nn.sigmoid(m)
    gt = (s > sv16).astype(jnp.float32)
    eq = (s == sv16).astype(jnp.float32)
    need = 16.0 - jnp.sum(gt, axis=1, keepdims=True)
    r_i = jax.lax.broadcasted_iota(jnp.int32, (_CW, _CW), 0)
    c_i = jax.lax.broadcasted_iota(jnp.int32, (_CW, _CW), 1)
    triu = (r_i < c_i).astype(jnp.bfloat16)
    off = jnp.zeros_like(need)
    for c in range(_NCH):
        sl = slice(c * _CW, (c + 1) * _CW)
        eq_c = eq[:, sl]
        rank_c = jax.lax.dot_general(eq_c.astype(jnp.bfloat16), triu,
                                     (((1,), (0,)), ((), ())),
                                     preferred_element_type=jnp.float32)
        keep_c = (gt[:, sl] + eq_c * (rank_c + off < need)) > 0.0
        out_ref[:, sl] = jnp.where(keep_c, s[:, sl], 0.0)
        off = off + jnp.sum(eq_c, axis=1, keepdims=True)


def _graph_kernel(qb_ref, kb_ref, qa_ref, ka_ref, sf_ref, sb_ref):
    qb = qb_ref[:].astype(jnp.bfloat16)
    kb = kb_ref[:].astype(jnp.bfloat16)
    qa = qa_ref[:].astype(jnp.bfloat16)
    ka = ka_ref[:].astype(jnp.bfloat16)
    ef = jax.lax.dot_general(qb, ka, (((1,), (1,)), ((), ())),
                             preferred_element_type=jnp.float32)
    eb = jax.lax.dot_general(kb, qa, (((1,), (1,)), ((), ())),
                             preferred_element_type=jnp.float32)
    _topk_masked(ef, _v16(ef), sf_ref)
    _topk_masked(eb, _v16(eb), sb_ref)


_LVL = 5


def _v16(e):
    work = [e[:, c * _CW:(c + 1) * _CW] for c in range(_NCH)]
    levels = []
    for lvl in range(_LVL):
        if lvl > 0:
            prev = levels[-1]
            work = [jnp.where(wc == prev, _NEG, wc) for wc in work]
        m = work[0]
        for c in range(1, _NCH):
            m = jnp.maximum(m, work[c])
        levels.append(m)
    v = None
    for i in range(_TOPK):
        g = levels[0]
        for lvl in range(1, _LVL):
            g = jnp.maximum(g, levels[lvl])
        v = jnp.max(g, axis=1, keepdims=True)
        if i != _TOPK - 1:
            levels = [jnp.where(lv == v, _NEG, lv) for lv in levels]
    return v


@functools.partial(jax.jit, static_argnames=("interpret",))
def _run(x_wav, W_shared, W_phi, b_phi, W_mu, W_Q, W_K, interpret=False):
    x2 = x_wav.reshape(_N, _T * _C)
    bphi2 = b_phi.reshape(1, _C)
    wmu2 = W_mu.reshape(1, _R)

    q, k = pl.pallas_call(
        _qk_kernel,
        grid=(_N // _NB_QK,),
        in_specs=[
            pl.BlockSpec((_NB_QK, _T * _C), lambda i: (i, 0)),
            pl.BlockSpec((_R, _C), lambda i: (0, 0)),
            pl.BlockSpec((_C, _R), lambda i: (0, 0)),
            pl.BlockSpec((1, _C), lambda i: (0, 0)),
            pl.BlockSpec((1, _R), lambda i: (0, 0)),
            pl.BlockSpec((_D, _C), lambda i: (0, 0)),
            pl.BlockSpec((_D, _C), lambda i: (0, 0)),
        ],
        out_specs=[
            pl.BlockSpec((_NB_QK, _D), lambda i: (i, 0)),
            pl.BlockSpec((_NB_QK, _D), lambda i: (i, 0)),
        ],
        out_shape=[
            jax.ShapeDtypeStruct((_N, _D), jnp.float32),
            jax.ShapeDtypeStruct((_N, _D), jnp.float32),
        ],
        interpret=interpret,
    )(x2, W_shared, W_phi, bphi2, wmu2, W_Q, W_K)

    sf, sb = pl.pallas_call(
        _graph_kernel,
        grid=(_N // _NB_G,),
        in_specs=[
            pl.BlockSpec((_NB_G, _D), lambda i: (i, 0)),
            pl.BlockSpec((_NB_G, _D), lambda i: (i, 0)),
            pl.BlockSpec((_N, _D), lambda i: (0, 0)),
            pl.BlockSpec((_N, _D), lambda i: (0, 0)),
        ],
        out_specs=[
            pl.BlockSpec((_NB_G, _N), lambda i: (i, 0)),
            pl.BlockSpec((_NB_G, _N), lambda i: (i, 0)),
        ],
        out_shape=[
            jax.ShapeDtypeStruct((_N, _N), jnp.float32),
            jax.ShapeDtypeStruct((_N, _N), jnp.float32),
        ],
        interpret=interpret,
    )(q, k, q, k)

    return sf[None], sb[None]


def kernel(x_wav, W_shared, W_phi, b_phi, W_mu, b_mu, W_Q, W_K):
    del b_mu
    return _run(x_wav, W_shared, W_phi, b_phi, W_mu, W_Q, W_K)
